# baseline (speedup 1.0000x reference)
import functools

import jax
import jax.numpy as jnp
import numpy as np
from jax import lax
from jax.experimental import pallas as pl
from jax.experimental.pallas import tpu as pltpu
from jax.experimental.pallas import tpu_sc as plsc

B_ = 4
N_ = 8192
M_ = 512
U_ = 32
MU_ = M_ * U_
G_ = 8
EPS_ = 1e-5

NC_ = 2
NS_ = 16
NW_ = NC_ * NS_
ROWS_ = B_ * MU_
RPW_ = ROWS_ // NW_
CH_ = 128
NCH_ = RPW_ // CH_
DT_ = 256


def _fps_body(c4_ref, ct_ref, cen_ref):
    Xr = c4_ref[0, 0]
    Yr = c4_ref[0, 1]
    Zr = c4_ref[0, 2]
    row0 = ct_ref[0, 0:1, :]
    cen_ref[0, 0:1, :] = row0
    dx = Xr - row0[:, 0:1]
    dy = Yr - row0[:, 1:2]
    dz = Zr - row0[:, 2:3]
    d0 = dx * dx + dy * dy + dz * dz
    lin = (lax.broadcasted_iota(jnp.int32, (8, N_ // 8), 0) * (N_ // 8)
           + lax.broadcasted_iota(jnp.int32, (8, N_ // 8), 1))

    def body(i, d):
        mx = jnp.max(d)
        far = jnp.min(jnp.where(d == mx, lin, jnp.int32(1 << 30)))
        rowf = ct_ref[0, pl.ds(far, 1), :]
        cen_ref[0, pl.ds(i, 1), :] = rowf
        ndx = Xr - rowf[:, 0:1]
        ndy = Yr - rowf[:, 1:2]
        ndz = Zr - rowf[:, 2:3]
        nd = ndx * ndx + ndy * ndy + ndz * ndz
        return jnp.minimum(d, nd)

    lax.fori_loop(1, M_, body, d0)


def _fps(coords4, coordsT):
    return pl.pallas_call(
        _fps_body,
        grid=(B_,),
        in_specs=[
            pl.BlockSpec((1, 3, 8, N_ // 8), lambda b: (b, 0, 0, 0)),
            pl.BlockSpec((1, N_, 3), lambda b: (b, 0, 0)),
        ],
        out_specs=pl.BlockSpec((1, M_, 3), lambda b: (b, 0, 0)),
        out_shape=jax.ShapeDtypeStruct((B_, M_, 3), jnp.float32),
    )(coords4, coordsT)


RC_ = 8
CHK_ = 128


NCHK_ = N_ // CHK_


def _knn_body(p_ref, c_ref, n_ref, d_ref, pv_ref, pi_ref):
    b = pl.program_id(0)
    cb = c_ref[0]
    X = p_ref[0, 0:1, :]
    Y = p_ref[0, 1:2, :]
    Z = p_ref[0, 2:3, :]
    cx = cb[:, 0:1]
    cy = cb[:, 1:2]
    cz = cb[:, 2:3]
    p2 = X * X + Y * Y + Z * Z
    c2 = cx * cx + cy * cy + cz * cz
    dot = jnp.dot(cb, p_ref[0], preferred_element_type=jnp.float32)
    d_ref[...] = ((c2 + p2) - 2.0 * dot).reshape(RC_, NCHK_, CHK_)
    pv_ref[...] = jnp.full((RC_, U_, NCHK_), jnp.inf, jnp.float32)
    iota128 = lax.broadcasted_iota(jnp.int32, (RC_, NCHK_, CHK_), 2)
    chunkoff = lax.broadcasted_iota(jnp.int32, (RC_, 1, NCHK_), 2) * CHK_

    def cond(state):
        r, done = state
        return (r < U_) & jnp.logical_not(done)

    def body(state):
        r, _ = state
        dv = d_ref[...]
        cm = jnp.min(dv, axis=2)
        pos = jnp.min(jnp.where(dv == cm[:, :, None], iota128,
                                jnp.int32(1 << 30)), axis=2)
        dv = jnp.where(iota128 == pos[:, :, None], jnp.float32(jnp.inf), dv)
        d_ref[...] = dv
        pv_ref[:, pl.ds(r, 1), :] = cm[:, None, :]
        pi_ref[:, pl.ds(r, 1), :] = pos[:, None, :] + chunkoff
        mnext = jnp.min(dv, axis=(1, 2))
        cnt = jnp.sum((pv_ref[...] < mnext[:, None, None]).astype(jnp.int32),
                      axis=(1, 2))
        return r + 1, jnp.all(cnt >= U_)

    lax.while_loop(cond, body, (jnp.int32(0), False))

    pv = pv_ref[...]
    pi = pi_ref[...]
    ids = []
    for _ in range(U_):
        m = jnp.min(pv, axis=(1, 2), keepdims=True)
        sel = jnp.min(jnp.where(pv == m, pi, jnp.int32(1 << 30)),
                      axis=(1, 2), keepdims=True)
        ids.append(sel[:, :, 0])
        pv = jnp.where(pi == sel, jnp.float32(jnp.inf), pv)
    nbr = jnp.concatenate(ids, axis=1) + b * N_
    n_ref[0] = nbr


def _knn(coords, centers):
    return pl.pallas_call(
        _knn_body,
        grid=(B_, M_ // RC_),
        in_specs=[
            pl.BlockSpec((1, 3, N_), lambda b, r: (b, 0, 0)),
            pl.BlockSpec((1, RC_, 3), lambda b, r: (b, r, 0)),
        ],
        out_specs=pl.BlockSpec((1, RC_, U_), lambda b, r: (b, r, 0)),
        out_shape=jax.ShapeDtypeStruct((B_, M_, U_), jnp.int32),
        scratch_shapes=[
            pltpu.VMEM((RC_, NCHK_, CHK_), jnp.float32),
            pltpu.VMEM((RC_, U_, NCHK_), jnp.float32),
            pltpu.VMEM((RC_, U_, NCHK_), jnp.int32),
        ],
    )(coords, centers)


@functools.cache
def _make_gather_sc():
    mesh = plsc.VectorSubcoreMesh(core_axis_name="c", subcore_axis_name="s")

    @functools.partial(
        pl.kernel,
        mesh=mesh,
        out_type=jax.ShapeDtypeStruct((ROWS_, DT_), jnp.float32),
        scratch_types=[
            pltpu.VMEM((CH_,), jnp.int32),
            pltpu.VMEM((CH_, DT_), jnp.float32),
            pltpu.SemaphoreType.DMA,
        ],
    )
    def _gather_sc(table, idx, g_out, idxv, rows, s1):
        wid = lax.axis_index("s") * NC_ + lax.axis_index("c")
        base = wid * RPW_

        def chunk(j, carry):
            off = base + j * CH_
            pltpu.sync_copy(idx.at[pl.ds(off, CH_)], idxv)
            pltpu.async_copy(table.at[idxv], rows, s1).wait()
            pltpu.sync_copy(rows, g_out.at[pl.ds(off, CH_)])
            return carry

        lax.fori_loop(0, NCH_, chunk, 0)

    return _gather_sc


RB_ = 512


def _layer0_body(g_ref, c_ref, wc_ref, wf_ref, b_ref,
                 y_ref, st_ref, tm_ref):
    r = pl.program_id(1)
    g = g_ref[0]
    xf = g[:, 0:128]
    c16 = c_ref[0]
    cexp = jnp.broadcast_to(c16[:, None, :],
                            (RB_ // U_, U_, 3)).reshape(RB_, 3)
    xc = g[:, 192:195] - cexp
    y = (jnp.dot(xc, wc_ref[...], preferred_element_type=jnp.float32)
         + jnp.dot(xf, wf_ref[...], preferred_element_type=jnp.float32)
         + b_ref[...])
    y_ref[0] = y
    s = jnp.sum(y, axis=0, keepdims=True)
    s2 = jnp.sum(y * y, axis=0, keepdims=True)
    st = jnp.concatenate([s, s2], axis=0)

    @pl.when(r == 0)
    def _():
        st_ref[0] = st

    @pl.when(r != 0)
    def _():
        st_ref[0] = st_ref[0] + st

    t = g[:, 128:192].reshape(RB_ // U_, U_, 64)
    tm_ref[0] = jnp.max(t, axis=1)


def _layer0(gath, centers, w0c, w0f, b0):
    return pl.pallas_call(
        _layer0_body,
        grid=(B_, MU_ // RB_),
        in_specs=[
            pl.BlockSpec((1, RB_, DT_), lambda b, r: (b, r, 0)),
            pl.BlockSpec((1, RB_ // U_, 3), lambda b, r: (b, r, 0)),
            pl.BlockSpec((3, 128), lambda b, r: (0, 0)),
            pl.BlockSpec((128, 128), lambda b, r: (0, 0)),
            pl.BlockSpec((1, 128), lambda b, r: (0, 0)),
        ],
        out_specs=[
            pl.BlockSpec((1, RB_, 128), lambda b, r: (b, r, 0)),
            pl.BlockSpec((1, 2, 128), lambda b, r: (b, 0, 0)),
            pl.BlockSpec((1, RB_ // U_, 64), lambda b, r: (b, r, 0)),
        ],
        out_shape=[
            jax.ShapeDtypeStruct((B_, MU_, 128), jnp.float32),
            jax.ShapeDtypeStruct((B_, 2, 128), jnp.float32),
            jax.ShapeDtypeStruct((B_, M_, 64), jnp.float32),
        ],
    )(gath, centers, w0c, w0f, b0)


def _norm_silu(y, st, gm_ref, gnw_ref, gnb_ref):
    mu = jnp.dot(st[0:1], gm_ref[...], preferred_element_type=jnp.float32,
                 precision=lax.Precision.HIGHEST)
    es2 = jnp.dot(st[1:2], gm_ref[...], preferred_element_type=jnp.float32,
                  precision=lax.Precision.HIGHEST)
    var = es2 - mu * mu
    inv = 1.0 / jnp.sqrt(var + EPS_)
    sc = inv * gnw_ref[...]
    sh = gnb_ref[...] - mu * sc
    z = y * sc + sh
    return z * (1.0 / (1.0 + jnp.exp(-z)))


def _mid_body(y_ref, st_in_ref, gm_ref, gnw_ref, gnb_ref, wt_ref, b_ref,
              yn_ref, st_ref):
    r = pl.program_id(1)
    z = _norm_silu(y_ref[0], st_in_ref[0], gm_ref, gnw_ref, gnb_ref)
    y2 = jnp.dot(z, wt_ref[...], preferred_element_type=jnp.float32) + b_ref[...]
    yn_ref[0] = y2
    s = jnp.sum(y2, axis=0, keepdims=True)
    s2 = jnp.sum(y2 * y2, axis=0, keepdims=True)
    st = jnp.concatenate([s, s2], axis=0)

    @pl.when(r == 0)
    def _():
        st_ref[0] = st

    @pl.when(r != 0)
    def _():
        st_ref[0] = st_ref[0] + st


def _mid_layer(y, stats, gm, gnw, gnb, wt, bias, cin, cout):
    return pl.pallas_call(
        _mid_body,
        grid=(B_, MU_ // RB_),
        in_specs=[
            pl.BlockSpec((1, RB_, cin), lambda b, r: (b, r, 0)),
            pl.BlockSpec((1, 2, cin), lambda b, r: (b, 0, 0)),
            pl.BlockSpec((cin, cin), lambda b, r: (0, 0)),
            pl.BlockSpec((1, cin), lambda b, r: (0, 0)),
            pl.BlockSpec((1, cin), lambda b, r: (0, 0)),
            pl.BlockSpec((cin, cout), lambda b, r: (0, 0)),
            pl.BlockSpec((1, cout), lambda b, r: (0, 0)),
        ],
        out_specs=[
            pl.BlockSpec((1, RB_, cout), lambda b, r: (b, r, 0)),
            pl.BlockSpec((1, 2, cout), lambda b, r: (b, 0, 0)),
        ],
        out_shape=[
            jax.ShapeDtypeStruct((B_, MU_, cout), jnp.float32),
            jax.ShapeDtypeStruct((B_, 2, cout), jnp.float32),
        ],
    )(y, stats, gm, gnw, gnb, wt, bias)


def _final_body(y_ref, st_in_ref, gm_ref, gnw_ref, gnb_ref, o_ref):
    z = _norm_silu(y_ref[0], st_in_ref[0], gm_ref, gnw_ref, gnb_ref)
    o_ref[0] = jnp.max(z.reshape(RB_ // U_, U_, 512), axis=1)


def _final(y, stats, gm, gnw, gnb):
    return pl.pallas_call(
        _final_body,
        grid=(B_, MU_ // RB_),
        in_specs=[
            pl.BlockSpec((1, RB_, 512), lambda b, r: (b, r, 0)),
            pl.BlockSpec((1, 2, 512), lambda b, r: (b, 0, 0)),
            pl.BlockSpec((512, 512), lambda b, r: (0, 0)),
            pl.BlockSpec((1, 512), lambda b, r: (0, 0)),
            pl.BlockSpec((1, 512), lambda b, r: (0, 0)),
        ],
        out_specs=pl.BlockSpec((1, RB_ // U_, 512), lambda b, r: (b, r, 0)),
        out_shape=jax.ShapeDtypeStruct((B_, M_, 512), jnp.float32),
    )(y, stats, gm, gnw, gnb)


def _gmat(c):
    g = c // G_
    return np.kron(np.eye(G_, dtype=np.float32),
                   np.ones((g, g), dtype=np.float32)) / np.float32(g * MU_)


_GM0 = _gmat(128)
_GM1 = _gmat(256)
_GM2 = _gmat(512)


def kernel(features, coords, temb, W0, b0, gnw0, gnb0, W1, b1, gnw1, gnb1,
           W2, b2, gnw2, gnb2):
    coordsT = jnp.transpose(coords, (0, 2, 1))
    coords4 = coords.reshape(B_, 3, 8, N_ // 8)

    centers = _fps(coords4, coordsT)
    nbr = _knn(coords, centers)

    featT = jnp.transpose(features, (0, 2, 1))
    tembT = jnp.transpose(temb, (0, 2, 1))
    table = jnp.concatenate(
        [featT, tembT, coordsT,
         jnp.zeros((B_, N_, DT_ - 195), jnp.float32)], axis=2,
    ).reshape(B_ * N_, DT_)

    gath = _make_gather_sc()(table, nbr.reshape(ROWS_))
    gath = gath.reshape(B_, MU_, DT_)

    w0c = jnp.transpose(W0[:, :3])
    w0f = jnp.transpose(W0[:, 3:])
    y0, st0, tmax = _layer0(gath, centers, w0c, w0f, b0.reshape(1, -1))
    y1, st1 = _mid_layer(y0, st0, _GM0, gnw0.reshape(1, -1),
                         gnb0.reshape(1, -1), jnp.transpose(W1),
                         b1.reshape(1, -1), 128, 256)
    y2, st2 = _mid_layer(y1, st1, _GM1, gnw1.reshape(1, -1),
                         gnb1.reshape(1, -1), jnp.transpose(W2),
                         b2.reshape(1, -1), 256, 512)
    out = _final(y2, st2, _GM2, gnw2.reshape(1, -1), gnb2.reshape(1, -1))

    return (jnp.transpose(out, (0, 2, 1)),
            jnp.transpose(centers, (0, 2, 1)),
            jnp.transpose(tmax, (0, 2, 1)))


# speedup vs baseline: 1.9581x; 1.9581x over previous
import functools

import jax
import jax.numpy as jnp
import numpy as np
from jax import lax
from jax.experimental import pallas as pl
from jax.experimental.pallas import tpu as pltpu
from jax.experimental.pallas import tpu_sc as plsc

B_ = 4
N_ = 8192
M_ = 512
U_ = 32
MU_ = M_ * U_
G_ = 8
EPS_ = 1e-5

NC_ = 2
NS_ = 16
NW_ = NC_ * NS_
ROWS_ = B_ * MU_
RPW_ = ROWS_ // NW_
CH_ = 128
NCH_ = RPW_ // CH_
DT_ = 256


def _fps_body(cv_ref, cen_ref):
    X4 = cv_ref[0].reshape(B_, 8, N_ // 8)
    Y4 = cv_ref[1].reshape(B_, 8, N_ // 8)
    Z4 = cv_ref[2].reshape(B_, 8, N_ // 8)
    lin3 = (lax.broadcasted_iota(jnp.int32, (B_, 8, N_ // 8), 1) * (N_ // 8)
            + lax.broadcasted_iota(jnp.int32, (B_, 8, N_ // 8), 2))

    xf = X4[:, 0:1, 0:1]
    yf = Y4[:, 0:1, 0:1]
    zf = Z4[:, 0:1, 0:1]
    cen_ref[:, 0:1, :] = jnp.concatenate([xf, yf, zf], axis=2)
    dx = X4 - xf
    dy = Y4 - yf
    dz = Z4 - zf
    d0 = dx * dx + dy * dy + dz * dz

    def body(i, d):
        mx = jnp.max(d, axis=(1, 2), keepdims=True)
        far = jnp.min(jnp.where(d == mx, lin3, jnp.int32(1 << 30)),
                      axis=(1, 2), keepdims=True)
        eqsel = lin3 == far
        ninf = jnp.float32(-jnp.inf)
        xf = jnp.max(jnp.where(eqsel, X4, ninf), axis=(1, 2), keepdims=True)
        yf = jnp.max(jnp.where(eqsel, Y4, ninf), axis=(1, 2), keepdims=True)
        zf = jnp.max(jnp.where(eqsel, Z4, ninf), axis=(1, 2), keepdims=True)
        cen_ref[:, pl.ds(i, 1), :] = jnp.concatenate([xf, yf, zf], axis=2)
        ndx = X4 - xf
        ndy = Y4 - yf
        ndz = Z4 - zf
        nd = ndx * ndx + ndy * ndy + ndz * ndz
        return jnp.minimum(d, nd)

    lax.fori_loop(1, M_, body, d0)


def _fps(coordsV):
    return pl.pallas_call(
        _fps_body,
        grid=(1,),
        in_specs=[pl.BlockSpec((3, B_ * 8, N_ // 8), lambda b: (0, 0, 0))],
        out_specs=pl.BlockSpec((B_, M_, 3), lambda b: (0, 0, 0)),
        out_shape=jax.ShapeDtypeStruct((B_, M_, 3), jnp.float32),
    )(coordsV)


RC_ = 16


def _knn_body(p_ref, c_ref, n_ref):
    b = pl.program_id(0)
    cb = c_ref[0]
    X = p_ref[0, 0:1, :]
    Y = p_ref[0, 1:2, :]
    Z = p_ref[0, 2:3, :]
    cx = cb[:, 0:1]
    cy = cb[:, 1:2]
    cz = cb[:, 2:3]
    p2 = X * X + Y * Y + Z * Z
    c2 = cx * cx + cy * cy + cz * cz
    dot = jnp.dot(cb, p_ref[0], preferred_element_type=jnp.float32)
    d = (c2 + p2) - 2.0 * dot
    lin = lax.broadcasted_iota(jnp.int32, (RC_, N_), 1)
    ids = []
    for _ in range(U_):
        m = jnp.min(d, axis=1, keepdims=True)
        idx = jnp.min(jnp.where(d == m, lin, jnp.int32(1 << 30)),
                      axis=1, keepdims=True)
        ids.append(idx)
        d = jnp.where(lin == idx, jnp.float32(jnp.inf), d)
    nbr = jnp.concatenate(ids, axis=1) + b * N_
    n_ref[0] = nbr


def _knn(coords, centers):
    return pl.pallas_call(
        _knn_body,
        grid=(B_, M_ // RC_),
        in_specs=[
            pl.BlockSpec((1, 3, N_), lambda b, r: (b, 0, 0)),
            pl.BlockSpec((1, RC_, 3), lambda b, r: (b, r, 0)),
        ],
        out_specs=pl.BlockSpec((1, RC_, U_), lambda b, r: (b, r, 0)),
        out_shape=jax.ShapeDtypeStruct((B_, M_, U_), jnp.int32),
    )(coords, centers)


@functools.cache
def _make_gather_sc():
    mesh = plsc.VectorSubcoreMesh(core_axis_name="c", subcore_axis_name="s")

    @functools.partial(
        pl.kernel,
        mesh=mesh,
        out_type=jax.ShapeDtypeStruct((ROWS_, DT_), jnp.float32),
        scratch_types=[
            pltpu.VMEM((CH_,), jnp.int32),
            pltpu.VMEM((CH_, DT_), jnp.float32),
            pltpu.SemaphoreType.DMA,
        ],
    )
    def _gather_sc(table, idx, g_out, idxv, rows, s1):
        wid = lax.axis_index("s") * NC_ + lax.axis_index("c")
        base = wid * RPW_

        def chunk(j, carry):
            off = base + j * CH_
            pltpu.sync_copy(idx.at[pl.ds(off, CH_)], idxv)
            pltpu.async_copy(table.at[idxv], rows, s1).wait()
            pltpu.sync_copy(rows, g_out.at[pl.ds(off, CH_)])
            return carry

        lax.fori_loop(0, NCH_, chunk, 0)

    return _gather_sc


RB_ = 512


def _layer0_body(g_ref, c_ref, wc_ref, wf_ref, b_ref,
                 y_ref, st_ref, tm_ref):
    r = pl.program_id(1)
    g = g_ref[0]
    xf = g[:, 0:128]
    c16 = c_ref[0]
    cexp = jnp.broadcast_to(c16[:, None, :],
                            (RB_ // U_, U_, 3)).reshape(RB_, 3)
    xc = g[:, 192:195] - cexp
    y = (jnp.dot(xc, wc_ref[...], preferred_element_type=jnp.float32)
         + jnp.dot(xf, wf_ref[...], preferred_element_type=jnp.float32)
         + b_ref[...])
    y_ref[0] = y
    s = jnp.sum(y, axis=0, keepdims=True)
    s2 = jnp.sum(y * y, axis=0, keepdims=True)
    st = jnp.concatenate([s, s2], axis=0)

    @pl.when(r == 0)
    def _():
        st_ref[0] = st

    @pl.when(r != 0)
    def _():
        st_ref[0] = st_ref[0] + st

    t = g[:, 128:192].reshape(RB_ // U_, U_, 64)
    tm_ref[0] = jnp.max(t, axis=1)


def _layer0(gath, centers, w0c, w0f, b0):
    return pl.pallas_call(
        _layer0_body,
        grid=(B_, MU_ // RB_),
        in_specs=[
            pl.BlockSpec((1, RB_, DT_), lambda b, r: (b, r, 0)),
            pl.BlockSpec((1, RB_ // U_, 3), lambda b, r: (b, r, 0)),
            pl.BlockSpec((3, 128), lambda b, r: (0, 0)),
            pl.BlockSpec((128, 128), lambda b, r: (0, 0)),
            pl.BlockSpec((1, 128), lambda b, r: (0, 0)),
        ],
        out_specs=[
            pl.BlockSpec((1, RB_, 128), lambda b, r: (b, r, 0)),
            pl.BlockSpec((1, 2, 128), lambda b, r: (b, 0, 0)),
            pl.BlockSpec((1, RB_ // U_, 64), lambda b, r: (b, r, 0)),
        ],
        out_shape=[
            jax.ShapeDtypeStruct((B_, MU_, 128), jnp.float32),
            jax.ShapeDtypeStruct((B_, 2, 128), jnp.float32),
            jax.ShapeDtypeStruct((B_, M_, 64), jnp.float32),
        ],
    )(gath, centers, w0c, w0f, b0)


def _norm_silu(y, st, gm_ref, gnw_ref, gnb_ref):
    mu = jnp.dot(st[0:1], gm_ref[...], preferred_element_type=jnp.float32,
                 precision=lax.Precision.HIGHEST)
    es2 = jnp.dot(st[1:2], gm_ref[...], preferred_element_type=jnp.float32,
                  precision=lax.Precision.HIGHEST)
    var = es2 - mu * mu
    inv = 1.0 / jnp.sqrt(var + EPS_)
    sc = inv * gnw_ref[...]
    sh = gnb_ref[...] - mu * sc
    z = y * sc + sh
    return z * (1.0 / (1.0 + jnp.exp(-z)))


def _mid_body(y_ref, st_in_ref, gm_ref, gnw_ref, gnb_ref, wt_ref, b_ref,
              yn_ref, st_ref):
    r = pl.program_id(1)
    z = _norm_silu(y_ref[0], st_in_ref[0], gm_ref, gnw_ref, gnb_ref)
    y2 = jnp.dot(z, wt_ref[...], preferred_element_type=jnp.float32) + b_ref[...]
    yn_ref[0] = y2
    s = jnp.sum(y2, axis=0, keepdims=True)
    s2 = jnp.sum(y2 * y2, axis=0, keepdims=True)
    st = jnp.concatenate([s, s2], axis=0)

    @pl.when(r == 0)
    def _():
        st_ref[0] = st

    @pl.when(r != 0)
    def _():
        st_ref[0] = st_ref[0] + st


def _mid_layer(y, stats, gm, gnw, gnb, wt, bias, cin, cout):
    return pl.pallas_call(
        _mid_body,
        grid=(B_, MU_ // RB_),
        in_specs=[
            pl.BlockSpec((1, RB_, cin), lambda b, r: (b, r, 0)),
            pl.BlockSpec((1, 2, cin), lambda b, r: (b, 0, 0)),
            pl.BlockSpec((cin, cin), lambda b, r: (0, 0)),
            pl.BlockSpec((1, cin), lambda b, r: (0, 0)),
            pl.BlockSpec((1, cin), lambda b, r: (0, 0)),
            pl.BlockSpec((cin, cout), lambda b, r: (0, 0)),
            pl.BlockSpec((1, cout), lambda b, r: (0, 0)),
        ],
        out_specs=[
            pl.BlockSpec((1, RB_, cout), lambda b, r: (b, r, 0)),
            pl.BlockSpec((1, 2, cout), lambda b, r: (b, 0, 0)),
        ],
        out_shape=[
            jax.ShapeDtypeStruct((B_, MU_, cout), jnp.float32),
            jax.ShapeDtypeStruct((B_, 2, cout), jnp.float32),
        ],
    )(y, stats, gm, gnw, gnb, wt, bias)


def _final_body(y_ref, st_in_ref, gm_ref, gnw_ref, gnb_ref, o_ref):
    z = _norm_silu(y_ref[0], st_in_ref[0], gm_ref, gnw_ref, gnb_ref)
    o_ref[0] = jnp.max(z.reshape(RB_ // U_, U_, 512), axis=1)


def _final(y, stats, gm, gnw, gnb):
    return pl.pallas_call(
        _final_body,
        grid=(B_, MU_ // RB_),
        in_specs=[
            pl.BlockSpec((1, RB_, 512), lambda b, r: (b, r, 0)),
            pl.BlockSpec((1, 2, 512), lambda b, r: (b, 0, 0)),
            pl.BlockSpec((512, 512), lambda b, r: (0, 0)),
            pl.BlockSpec((1, 512), lambda b, r: (0, 0)),
            pl.BlockSpec((1, 512), lambda b, r: (0, 0)),
        ],
        out_specs=pl.BlockSpec((1, RB_ // U_, 512), lambda b, r: (b, r, 0)),
        out_shape=jax.ShapeDtypeStruct((B_, M_, 512), jnp.float32),
    )(y, stats, gm, gnw, gnb)


def _gmat(c):
    g = c // G_
    return np.kron(np.eye(G_, dtype=np.float32),
                   np.ones((g, g), dtype=np.float32)) / np.float32(g * MU_)


_GM0 = _gmat(128)
_GM1 = _gmat(256)
_GM2 = _gmat(512)


def kernel(features, coords, temb, W0, b0, gnw0, gnb0, W1, b1, gnw1, gnb1,
           W2, b2, gnw2, gnb2):
    coordsT = jnp.transpose(coords, (0, 2, 1))
    coordsV = jnp.transpose(coords.reshape(B_, 3, 8, N_ // 8),
                            (1, 0, 2, 3)).reshape(3, B_ * 8, N_ // 8)

    centers = _fps(coordsV)
    nbr = _knn(coords, centers)

    featT = jnp.transpose(features, (0, 2, 1))
    tembT = jnp.transpose(temb, (0, 2, 1))
    table = jnp.concatenate(
        [featT, tembT, coordsT,
         jnp.zeros((B_, N_, DT_ - 195), jnp.float32)], axis=2,
    ).reshape(B_ * N_, DT_)

    gath = _make_gather_sc()(table, nbr.reshape(ROWS_))
    gath = gath.reshape(B_, MU_, DT_)

    w0c = jnp.transpose(W0[:, :3])
    w0f = jnp.transpose(W0[:, 3:])
    y0, st0, tmax = _layer0(gath, centers, w0c, w0f, b0.reshape(1, -1))
    y1, st1 = _mid_layer(y0, st0, _GM0, gnw0.reshape(1, -1),
                         gnb0.reshape(1, -1), jnp.transpose(W1),
                         b1.reshape(1, -1), 128, 256)
    y2, st2 = _mid_layer(y1, st1, _GM1, gnw1.reshape(1, -1),
                         gnb1.reshape(1, -1), jnp.transpose(W2),
                         b2.reshape(1, -1), 256, 512)
    out = _final(y2, st2, _GM2, gnw2.reshape(1, -1), gnb2.reshape(1, -1))

    return (jnp.transpose(out, (0, 2, 1)),
            jnp.transpose(centers, (0, 2, 1)),
            jnp.transpose(tmax, (0, 2, 1)))


# speedup vs baseline: 2.5295x; 1.2918x over previous
import functools

import jax
import jax.numpy as jnp
import numpy as np
from jax import lax
from jax.experimental import pallas as pl
from jax.experimental.pallas import tpu as pltpu
from jax.experimental.pallas import tpu_sc as plsc

B_ = 4
N_ = 8192
M_ = 512
U_ = 32
MU_ = M_ * U_
G_ = 8
EPS_ = 1e-5

NC_ = 2
NS_ = 16
NW_ = NC_ * NS_
ROWS_ = B_ * MU_
RPW_ = ROWS_ // NW_
CH_ = 128
NCH_ = RPW_ // CH_
DT_ = 256


def _fps_body(cv_ref, cen_ref):
    X4 = cv_ref[0].reshape(B_, 8, N_ // 8)
    Y4 = cv_ref[1].reshape(B_, 8, N_ // 8)
    Z4 = cv_ref[2].reshape(B_, 8, N_ // 8)
    lin3 = (lax.broadcasted_iota(jnp.int32, (B_, 8, N_ // 8), 1) * (N_ // 8)
            + lax.broadcasted_iota(jnp.int32, (B_, 8, N_ // 8), 2))

    xf = X4[:, 0:1, 0:1]
    yf = Y4[:, 0:1, 0:1]
    zf = Z4[:, 0:1, 0:1]
    cen_ref[:, 0:1, :] = jnp.concatenate([xf, yf, zf], axis=2)
    dx = X4 - xf
    dy = Y4 - yf
    dz = Z4 - zf
    d0 = dx * dx + dy * dy + dz * dz

    def body(i, d):
        mx = jnp.max(d, axis=(1, 2), keepdims=True)
        far = jnp.min(jnp.where(d == mx, lin3, jnp.int32(1 << 30)),
                      axis=(1, 2), keepdims=True)
        eqsel = lin3 == far
        ninf = jnp.float32(-jnp.inf)
        xf = jnp.max(jnp.where(eqsel, X4, ninf), axis=(1, 2), keepdims=True)
        yf = jnp.max(jnp.where(eqsel, Y4, ninf), axis=(1, 2), keepdims=True)
        zf = jnp.max(jnp.where(eqsel, Z4, ninf), axis=(1, 2), keepdims=True)
        cen_ref[:, pl.ds(i, 1), :] = jnp.concatenate([xf, yf, zf], axis=2)
        ndx = X4 - xf
        ndy = Y4 - yf
        ndz = Z4 - zf
        nd = ndx * ndx + ndy * ndy + ndz * ndz
        return jnp.minimum(d, nd)

    lax.fori_loop(1, M_, body, d0)


def _fps(coordsV):
    return pl.pallas_call(
        _fps_body,
        grid=(1,),
        in_specs=[pl.BlockSpec((3, B_ * 8, N_ // 8), lambda b: (0, 0, 0))],
        out_specs=pl.BlockSpec((B_, M_, 3), lambda b: (0, 0, 0)),
        out_shape=jax.ShapeDtypeStruct((B_, M_, 3), jnp.float32),
    )(coordsV)


RC_ = 32


def _knn_body(p_ref, c_ref, n_ref):
    b = pl.program_id(0)
    cb = c_ref[0]
    X = p_ref[0, 0:1, :]
    Y = p_ref[0, 1:2, :]
    Z = p_ref[0, 2:3, :]
    cx = cb[:, 0:1]
    cy = cb[:, 1:2]
    cz = cb[:, 2:3]
    p2 = X * X + Y * Y + Z * Z
    c2 = cx * cx + cy * cy + cz * cz
    dot = jnp.dot(cb, p_ref[0], preferred_element_type=jnp.float32)
    d = (c2 + p2) - 2.0 * dot
    lin = lax.broadcasted_iota(jnp.int32, (RC_, N_), 1)
    ids = []
    for _ in range(U_):
        m = jnp.min(d, axis=1, keepdims=True)
        idx = jnp.min(jnp.where(d == m, lin, jnp.int32(1 << 30)),
                      axis=1, keepdims=True)
        ids.append(idx)
        d = jnp.where(lin == idx, jnp.float32(jnp.inf), d)
    nbr = jnp.concatenate(ids, axis=1) + b * N_
    n_ref[0] = nbr


def _knn(coords, centers):
    return pl.pallas_call(
        _knn_body,
        grid=(B_, M_ // RC_),
        in_specs=[
            pl.BlockSpec((1, 3, N_), lambda b, r: (b, 0, 0)),
            pl.BlockSpec((1, RC_, 3), lambda b, r: (b, r, 0)),
        ],
        out_specs=pl.BlockSpec((1, RC_, U_), lambda b, r: (b, r, 0)),
        out_shape=jax.ShapeDtypeStruct((B_, M_, U_), jnp.int32),
    )(coords, centers)


@functools.cache
def _make_gather_sc():
    mesh = plsc.VectorSubcoreMesh(core_axis_name="c", subcore_axis_name="s")

    @functools.partial(
        pl.kernel,
        mesh=mesh,
        out_type=jax.ShapeDtypeStruct((ROWS_, DT_), jnp.float32),
        scratch_types=[
            pltpu.VMEM((CH_,), jnp.int32),
            pltpu.VMEM((CH_, DT_), jnp.float32),
            pltpu.SemaphoreType.DMA,
        ],
    )
    def _gather_sc(table, idx, g_out, idxv, rows, s1):
        wid = lax.axis_index("s") * NC_ + lax.axis_index("c")
        base = wid * RPW_

        def chunk(j, carry):
            off = base + j * CH_
            pltpu.sync_copy(idx.at[pl.ds(off, CH_)], idxv)
            pltpu.async_copy(table.at[idxv], rows, s1).wait()
            pltpu.sync_copy(rows, g_out.at[pl.ds(off, CH_)])
            return carry

        lax.fori_loop(0, NCH_, chunk, 0)

    return _gather_sc


RB_ = 512


def _layer0_body(g_ref, c_ref, wc_ref, wf_ref, b_ref,
                 y_ref, st_ref, tm_ref):
    r = pl.program_id(1)
    g = g_ref[0]
    xf = g[:, 0:128]
    c16 = c_ref[0]
    cexp = jnp.broadcast_to(c16[:, None, :],
                            (RB_ // U_, U_, 3)).reshape(RB_, 3)
    xc = g[:, 192:195] - cexp
    y = (jnp.dot(xc, wc_ref[...], preferred_element_type=jnp.float32)
         + jnp.dot(xf, wf_ref[...], preferred_element_type=jnp.float32)
         + b_ref[...])
    y_ref[0] = y
    s = jnp.sum(y, axis=0, keepdims=True)
    s2 = jnp.sum(y * y, axis=0, keepdims=True)
    st = jnp.concatenate([s, s2], axis=0)

    @pl.when(r == 0)
    def _():
        st_ref[0] = st

    @pl.when(r != 0)
    def _():
        st_ref[0] = st_ref[0] + st

    t = g[:, 128:192].reshape(RB_ // U_, U_, 64)
    tm_ref[0] = jnp.max(t, axis=1)


def _layer0(gath, centers, w0c, w0f, b0):
    return pl.pallas_call(
        _layer0_body,
        grid=(B_, MU_ // RB_),
        in_specs=[
            pl.BlockSpec((1, RB_, DT_), lambda b, r: (b, r, 0)),
            pl.BlockSpec((1, RB_ // U_, 3), lambda b, r: (b, r, 0)),
            pl.BlockSpec((3, 128), lambda b, r: (0, 0)),
            pl.BlockSpec((128, 128), lambda b, r: (0, 0)),
            pl.BlockSpec((1, 128), lambda b, r: (0, 0)),
        ],
        out_specs=[
            pl.BlockSpec((1, RB_, 128), lambda b, r: (b, r, 0)),
            pl.BlockSpec((1, 2, 128), lambda b, r: (b, 0, 0)),
            pl.BlockSpec((1, RB_ // U_, 64), lambda b, r: (b, r, 0)),
        ],
        out_shape=[
            jax.ShapeDtypeStruct((B_, MU_, 128), jnp.float32),
            jax.ShapeDtypeStruct((B_, 2, 128), jnp.float32),
            jax.ShapeDtypeStruct((B_, M_, 64), jnp.float32),
        ],
    )(gath, centers, w0c, w0f, b0)


def _norm_silu(y, st, gm_ref, gnw_ref, gnb_ref):
    mu = jnp.dot(st[0:1], gm_ref[...], preferred_element_type=jnp.float32,
                 precision=lax.Precision.HIGHEST)
    es2 = jnp.dot(st[1:2], gm_ref[...], preferred_element_type=jnp.float32,
                  precision=lax.Precision.HIGHEST)
    var = es2 - mu * mu
    inv = 1.0 / jnp.sqrt(var + EPS_)
    sc = inv * gnw_ref[...]
    sh = gnb_ref[...] - mu * sc
    z = y * sc + sh
    return z * (1.0 / (1.0 + jnp.exp(-z)))


def _mid_body(y_ref, st_in_ref, gm_ref, gnw_ref, gnb_ref, wt_ref, b_ref,
              yn_ref, st_ref):
    r = pl.program_id(1)
    z = _norm_silu(y_ref[0], st_in_ref[0], gm_ref, gnw_ref, gnb_ref)
    y2 = jnp.dot(z, wt_ref[...], preferred_element_type=jnp.float32) + b_ref[...]
    yn_ref[0] = y2
    s = jnp.sum(y2, axis=0, keepdims=True)
    s2 = jnp.sum(y2 * y2, axis=0, keepdims=True)
    st = jnp.concatenate([s, s2], axis=0)

    @pl.when(r == 0)
    def _():
        st_ref[0] = st

    @pl.when(r != 0)
    def _():
        st_ref[0] = st_ref[0] + st


def _mid_layer(y, stats, gm, gnw, gnb, wt, bias, cin, cout):
    return pl.pallas_call(
        _mid_body,
        grid=(B_, MU_ // RB_),
        in_specs=[
            pl.BlockSpec((1, RB_, cin), lambda b, r: (b, r, 0)),
            pl.BlockSpec((1, 2, cin), lambda b, r: (b, 0, 0)),
            pl.BlockSpec((cin, cin), lambda b, r: (0, 0)),
            pl.BlockSpec((1, cin), lambda b, r: (0, 0)),
            pl.BlockSpec((1, cin), lambda b, r: (0, 0)),
            pl.BlockSpec((cin, cout), lambda b, r: (0, 0)),
            pl.BlockSpec((1, cout), lambda b, r: (0, 0)),
        ],
        out_specs=[
            pl.BlockSpec((1, RB_, cout), lambda b, r: (b, r, 0)),
            pl.BlockSpec((1, 2, cout), lambda b, r: (b, 0, 0)),
        ],
        out_shape=[
            jax.ShapeDtypeStruct((B_, MU_, cout), jnp.float32),
            jax.ShapeDtypeStruct((B_, 2, cout), jnp.float32),
        ],
    )(y, stats, gm, gnw, gnb, wt, bias)


def _final_body(y_ref, st_in_ref, gm_ref, gnw_ref, gnb_ref, o_ref):
    z = _norm_silu(y_ref[0], st_in_ref[0], gm_ref, gnw_ref, gnb_ref)
    o_ref[0] = jnp.max(z.reshape(RB_ // U_, U_, 512), axis=1)


def _final(y, stats, gm, gnw, gnb):
    return pl.pallas_call(
        _final_body,
        grid=(B_, MU_ // RB_),
        in_specs=[
            pl.BlockSpec((1, RB_, 512), lambda b, r: (b, r, 0)),
            pl.BlockSpec((1, 2, 512), lambda b, r: (b, 0, 0)),
            pl.BlockSpec((512, 512), lambda b, r: (0, 0)),
            pl.BlockSpec((1, 512), lambda b, r: (0, 0)),
            pl.BlockSpec((1, 512), lambda b, r: (0, 0)),
        ],
        out_specs=pl.BlockSpec((1, RB_ // U_, 512), lambda b, r: (b, r, 0)),
        out_shape=jax.ShapeDtypeStruct((B_, M_, 512), jnp.float32),
    )(y, stats, gm, gnw, gnb)


def _gmat(c):
    g = c // G_
    return np.kron(np.eye(G_, dtype=np.float32),
                   np.ones((g, g), dtype=np.float32)) / np.float32(g * MU_)


_GM0 = _gmat(128)
_GM1 = _gmat(256)
_GM2 = _gmat(512)


def kernel(features, coords, temb, W0, b0, gnw0, gnb0, W1, b1, gnw1, gnb1,
           W2, b2, gnw2, gnb2):
    coordsT = jnp.transpose(coords, (0, 2, 1))
    coordsV = jnp.transpose(coords.reshape(B_, 3, 8, N_ // 8),
                            (1, 0, 2, 3)).reshape(3, B_ * 8, N_ // 8)

    centers = _fps(coordsV)
    nbr = _knn(coords, centers)

    featT = jnp.transpose(features, (0, 2, 1))
    tembT = jnp.transpose(temb, (0, 2, 1))
    table = jnp.concatenate(
        [featT, tembT, coordsT,
         jnp.zeros((B_, N_, DT_ - 195), jnp.float32)], axis=2,
    ).reshape(B_ * N_, DT_)

    gath = _make_gather_sc()(table, nbr.reshape(ROWS_))
    gath = gath.reshape(B_, MU_, DT_)

    w0c = jnp.transpose(W0[:, :3])
    w0f = jnp.transpose(W0[:, 3:])
    y0, st0, tmax = _layer0(gath, centers, w0c, w0f, b0.reshape(1, -1))
    y1, st1 = _mid_layer(y0, st0, _GM0, gnw0.reshape(1, -1),
                         gnb0.reshape(1, -1), jnp.transpose(W1),
                         b1.reshape(1, -1), 128, 256)
    y2, st2 = _mid_layer(y1, st1, _GM1, gnw1.reshape(1, -1),
                         gnb1.reshape(1, -1), jnp.transpose(W2),
                         b2.reshape(1, -1), 256, 512)
    out = _final(y2, st2, _GM2, gnw2.reshape(1, -1), gnb2.reshape(1, -1))

    return (jnp.transpose(out, (0, 2, 1)),
            jnp.transpose(centers, (0, 2, 1)),
            jnp.transpose(tmax, (0, 2, 1)))


# speedup vs baseline: 2.8345x; 1.1206x over previous
import functools

import jax
import jax.numpy as jnp
import numpy as np
from jax import lax
from jax.experimental import pallas as pl
from jax.experimental.pallas import tpu as pltpu
from jax.experimental.pallas import tpu_sc as plsc

B_ = 4
N_ = 8192
M_ = 512
U_ = 32
MU_ = M_ * U_
G_ = 8
EPS_ = 1e-5

NC_ = 2
NS_ = 16
NW_ = NC_ * NS_
ROWS_ = B_ * MU_
RPW_ = ROWS_ // NW_
CH_ = 128
NCH_ = RPW_ // CH_
DT_ = 256


def _fps_body(cv_ref, cen_ref):
    X4 = cv_ref[0].reshape(B_, 8, N_ // 8)
    Y4 = cv_ref[1].reshape(B_, 8, N_ // 8)
    Z4 = cv_ref[2].reshape(B_, 8, N_ // 8)
    lin3 = (lax.broadcasted_iota(jnp.int32, (B_, 8, N_ // 8), 1) * (N_ // 8)
            + lax.broadcasted_iota(jnp.int32, (B_, 8, N_ // 8), 2))

    xf = X4[:, 0:1, 0:1]
    yf = Y4[:, 0:1, 0:1]
    zf = Z4[:, 0:1, 0:1]
    cen_ref[:, 0:1, :] = jnp.concatenate([xf, yf, zf], axis=2)
    dx = X4 - xf
    dy = Y4 - yf
    dz = Z4 - zf
    d0 = dx * dx + dy * dy + dz * dz

    def body(i, d):
        mx = jnp.max(d, axis=(1, 2), keepdims=True)
        far = jnp.min(jnp.where(d == mx, lin3, jnp.int32(1 << 30)),
                      axis=(1, 2), keepdims=True)
        eqsel = lin3 == far
        ninf = jnp.float32(-jnp.inf)
        xf = jnp.max(jnp.where(eqsel, X4, ninf), axis=(1, 2), keepdims=True)
        yf = jnp.max(jnp.where(eqsel, Y4, ninf), axis=(1, 2), keepdims=True)
        zf = jnp.max(jnp.where(eqsel, Z4, ninf), axis=(1, 2), keepdims=True)
        cen_ref[:, pl.ds(i, 1), :] = jnp.concatenate([xf, yf, zf], axis=2)
        ndx = X4 - xf
        ndy = Y4 - yf
        ndz = Z4 - zf
        nd = ndx * ndx + ndy * ndy + ndz * ndz
        return jnp.minimum(d, nd)

    lax.fori_loop(1, M_, body, d0)


def _fps(coordsV):
    return pl.pallas_call(
        _fps_body,
        grid=(1,),
        in_specs=[pl.BlockSpec((3, B_ * 8, N_ // 8), lambda b: (0, 0, 0))],
        out_specs=pl.BlockSpec((B_, M_, 3), lambda b: (0, 0, 0)),
        out_shape=jax.ShapeDtypeStruct((B_, M_, 3), jnp.float32),
    )(coordsV)


RC_ = 64


def _knn_body(p_ref, c_ref, n_ref):
    b = pl.program_id(0)
    cb = c_ref[0]
    X = p_ref[0, 0:1, :]
    Y = p_ref[0, 1:2, :]
    Z = p_ref[0, 2:3, :]
    cx = cb[:, 0:1]
    cy = cb[:, 1:2]
    cz = cb[:, 2:3]
    p2 = X * X + Y * Y + Z * Z
    c2 = cx * cx + cy * cy + cz * cz
    dot = jnp.dot(cb, p_ref[0], preferred_element_type=jnp.float32)
    d = (c2 + p2) - 2.0 * dot
    lin = lax.broadcasted_iota(jnp.int32, (RC_, N_), 1)
    ids = []
    for _ in range(U_):
        m = jnp.min(d, axis=1, keepdims=True)
        idx = jnp.min(jnp.where(d == m, lin, jnp.int32(1 << 30)),
                      axis=1, keepdims=True)
        ids.append(idx)
        d = jnp.where(lin == idx, jnp.float32(jnp.inf), d)
    nbr = jnp.concatenate(ids, axis=1) + b * N_
    n_ref[0] = nbr


def _knn(coords, centers):
    return pl.pallas_call(
        _knn_body,
        grid=(B_, M_ // RC_),
        in_specs=[
            pl.BlockSpec((1, 3, N_), lambda b, r: (b, 0, 0)),
            pl.BlockSpec((1, RC_, 3), lambda b, r: (b, r, 0)),
        ],
        out_specs=pl.BlockSpec((1, RC_, U_), lambda b, r: (b, r, 0)),
        out_shape=jax.ShapeDtypeStruct((B_, M_, U_), jnp.int32),
    )(coords, centers)


@functools.cache
def _make_gather_sc():
    mesh = plsc.VectorSubcoreMesh(core_axis_name="c", subcore_axis_name="s")

    @functools.partial(
        pl.kernel,
        mesh=mesh,
        out_type=jax.ShapeDtypeStruct((ROWS_, DT_), jnp.float32),
        scratch_types=[
            pltpu.VMEM((CH_,), jnp.int32),
            pltpu.VMEM((CH_, DT_), jnp.float32),
            pltpu.SemaphoreType.DMA,
        ],
    )
    def _gather_sc(table, idx, g_out, idxv, rows, s1):
        wid = lax.axis_index("s") * NC_ + lax.axis_index("c")
        base = wid * RPW_

        def chunk(j, carry):
            off = base + j * CH_
            pltpu.sync_copy(idx.at[pl.ds(off, CH_)], idxv)
            pltpu.async_copy(table.at[idxv], rows, s1).wait()
            pltpu.sync_copy(rows, g_out.at[pl.ds(off, CH_)])
            return carry

        lax.fori_loop(0, NCH_, chunk, 0)

    return _gather_sc


RB_ = 512


def _layer0_body(g_ref, c_ref, wc_ref, wf_ref, b_ref,
                 y_ref, st_ref, tm_ref):
    r = pl.program_id(1)
    g = g_ref[0]
    xf = g[:, 0:128]
    c16 = c_ref[0]
    cexp = jnp.broadcast_to(c16[:, None, :],
                            (RB_ // U_, U_, 3)).reshape(RB_, 3)
    xc = g[:, 192:195] - cexp
    y = (jnp.dot(xc, wc_ref[...], preferred_element_type=jnp.float32)
         + jnp.dot(xf, wf_ref[...], preferred_element_type=jnp.float32)
         + b_ref[...])
    y_ref[0] = y
    s = jnp.sum(y, axis=0, keepdims=True)
    s2 = jnp.sum(y * y, axis=0, keepdims=True)
    st = jnp.concatenate([s, s2], axis=0)

    @pl.when(r == 0)
    def _():
        st_ref[0] = st

    @pl.when(r != 0)
    def _():
        st_ref[0] = st_ref[0] + st

    t = g[:, 128:192].reshape(RB_ // U_, U_, 64)
    tm_ref[0] = jnp.max(t, axis=1)


def _layer0(gath, centers, w0c, w0f, b0):
    return pl.pallas_call(
        _layer0_body,
        grid=(B_, MU_ // RB_),
        in_specs=[
            pl.BlockSpec((1, RB_, DT_), lambda b, r: (b, r, 0)),
            pl.BlockSpec((1, RB_ // U_, 3), lambda b, r: (b, r, 0)),
            pl.BlockSpec((3, 128), lambda b, r: (0, 0)),
            pl.BlockSpec((128, 128), lambda b, r: (0, 0)),
            pl.BlockSpec((1, 128), lambda b, r: (0, 0)),
        ],
        out_specs=[
            pl.BlockSpec((1, RB_, 128), lambda b, r: (b, r, 0)),
            pl.BlockSpec((1, 2, 128), lambda b, r: (b, 0, 0)),
            pl.BlockSpec((1, RB_ // U_, 64), lambda b, r: (b, r, 0)),
        ],
        out_shape=[
            jax.ShapeDtypeStruct((B_, MU_, 128), jnp.float32),
            jax.ShapeDtypeStruct((B_, 2, 128), jnp.float32),
            jax.ShapeDtypeStruct((B_, M_, 64), jnp.float32),
        ],
    )(gath, centers, w0c, w0f, b0)


def _norm_silu(y, st, gm_ref, gnw_ref, gnb_ref):
    mu = jnp.dot(st[0:1], gm_ref[...], preferred_element_type=jnp.float32,
                 precision=lax.Precision.HIGHEST)
    es2 = jnp.dot(st[1:2], gm_ref[...], preferred_element_type=jnp.float32,
                  precision=lax.Precision.HIGHEST)
    var = es2 - mu * mu
    inv = 1.0 / jnp.sqrt(var + EPS_)
    sc = inv * gnw_ref[...]
    sh = gnb_ref[...] - mu * sc
    z = y * sc + sh
    return z * (1.0 / (1.0 + jnp.exp(-z)))


def _mid_body(y_ref, st_in_ref, gm_ref, gnw_ref, gnb_ref, wt_ref, b_ref,
              yn_ref, st_ref):
    r = pl.program_id(1)
    z = _norm_silu(y_ref[0], st_in_ref[0], gm_ref, gnw_ref, gnb_ref)
    y2 = jnp.dot(z, wt_ref[...], preferred_element_type=jnp.float32) + b_ref[...]
    yn_ref[0] = y2
    s = jnp.sum(y2, axis=0, keepdims=True)
    s2 = jnp.sum(y2 * y2, axis=0, keepdims=True)
    st = jnp.concatenate([s, s2], axis=0)

    @pl.when(r == 0)
    def _():
        st_ref[0] = st

    @pl.when(r != 0)
    def _():
        st_ref[0] = st_ref[0] + st


def _mid_layer(y, stats, gm, gnw, gnb, wt, bias, cin, cout):
    return pl.pallas_call(
        _mid_body,
        grid=(B_, MU_ // RB_),
        in_specs=[
            pl.BlockSpec((1, RB_, cin), lambda b, r: (b, r, 0)),
            pl.BlockSpec((1, 2, cin), lambda b, r: (b, 0, 0)),
            pl.BlockSpec((cin, cin), lambda b, r: (0, 0)),
            pl.BlockSpec((1, cin), lambda b, r: (0, 0)),
            pl.BlockSpec((1, cin), lambda b, r: (0, 0)),
            pl.BlockSpec((cin, cout), lambda b, r: (0, 0)),
            pl.BlockSpec((1, cout), lambda b, r: (0, 0)),
        ],
        out_specs=[
            pl.BlockSpec((1, RB_, cout), lambda b, r: (b, r, 0)),
            pl.BlockSpec((1, 2, cout), lambda b, r: (b, 0, 0)),
        ],
        out_shape=[
            jax.ShapeDtypeStruct((B_, MU_, cout), jnp.float32),
            jax.ShapeDtypeStruct((B_, 2, cout), jnp.float32),
        ],
    )(y, stats, gm, gnw, gnb, wt, bias)


def _final_body(y_ref, st_in_ref, gm_ref, gnw_ref, gnb_ref, o_ref):
    z = _norm_silu(y_ref[0], st_in_ref[0], gm_ref, gnw_ref, gnb_ref)
    o_ref[0] = jnp.max(z.reshape(RB_ // U_, U_, 512), axis=1)


def _final(y, stats, gm, gnw, gnb):
    return pl.pallas_call(
        _final_body,
        grid=(B_, MU_ // RB_),
        in_specs=[
            pl.BlockSpec((1, RB_, 512), lambda b, r: (b, r, 0)),
            pl.BlockSpec((1, 2, 512), lambda b, r: (b, 0, 0)),
            pl.BlockSpec((512, 512), lambda b, r: (0, 0)),
            pl.BlockSpec((1, 512), lambda b, r: (0, 0)),
            pl.BlockSpec((1, 512), lambda b, r: (0, 0)),
        ],
        out_specs=pl.BlockSpec((1, RB_ // U_, 512), lambda b, r: (b, r, 0)),
        out_shape=jax.ShapeDtypeStruct((B_, M_, 512), jnp.float32),
    )(y, stats, gm, gnw, gnb)


def _gmat(c):
    g = c // G_
    return np.kron(np.eye(G_, dtype=np.float32),
                   np.ones((g, g), dtype=np.float32)) / np.float32(g * MU_)


_GM0 = _gmat(128)
_GM1 = _gmat(256)
_GM2 = _gmat(512)


def kernel(features, coords, temb, W0, b0, gnw0, gnb0, W1, b1, gnw1, gnb1,
           W2, b2, gnw2, gnb2):
    coordsT = jnp.transpose(coords, (0, 2, 1))
    coordsV = jnp.transpose(coords.reshape(B_, 3, 8, N_ // 8),
                            (1, 0, 2, 3)).reshape(3, B_ * 8, N_ // 8)

    centers = _fps(coordsV)
    nbr = _knn(coords, centers)

    featT = jnp.transpose(features, (0, 2, 1))
    tembT = jnp.transpose(temb, (0, 2, 1))
    table = jnp.concatenate(
        [featT, tembT, coordsT,
         jnp.zeros((B_, N_, DT_ - 195), jnp.float32)], axis=2,
    ).reshape(B_ * N_, DT_)

    gath = _make_gather_sc()(table, nbr.reshape(ROWS_))
    gath = gath.reshape(B_, MU_, DT_)

    w0c = jnp.transpose(W0[:, :3])
    w0f = jnp.transpose(W0[:, 3:])
    y0, st0, tmax = _layer0(gath, centers, w0c, w0f, b0.reshape(1, -1))
    y1, st1 = _mid_layer(y0, st0, _GM0, gnw0.reshape(1, -1),
                         gnb0.reshape(1, -1), jnp.transpose(W1),
                         b1.reshape(1, -1), 128, 256)
    y2, st2 = _mid_layer(y1, st1, _GM1, gnw1.reshape(1, -1),
                         gnb1.reshape(1, -1), jnp.transpose(W2),
                         b2.reshape(1, -1), 256, 512)
    out = _final(y2, st2, _GM2, gnw2.reshape(1, -1), gnb2.reshape(1, -1))

    return (jnp.transpose(out, (0, 2, 1)),
            jnp.transpose(centers, (0, 2, 1)),
            jnp.transpose(tmax, (0, 2, 1)))
